# baseline (speedup 1.0000x reference)
import functools

import jax
import jax.numpy as jnp
from jax import lax
from jax.experimental import pallas as pl
from jax.experimental.pallas import tpu as pltpu
from jax.experimental.pallas import tpu_sc as plsc

NC = 2
NS = 16
NW = NC * NS
K = 128
ZR = 32
UNROLL = 6


def _sc_spmm(xf, colf, rowp, valp, *, n_pad, steps, nb, c):
    rows_per_tile = n_pad // NS
    n_iter = steps // UNROLL

    mesh = plsc.VectorSubcoreMesh(core_axis_name="c", subcore_axis_name="s")

    @functools.partial(
        pl.kernel,
        out_type=jax.ShapeDtypeStruct((NC, nb, n_pad, c), jnp.float32),
        mesh=mesh,
        compiler_params=pltpu.CompilerParams(needs_layout_passes=False),
        scratch_types=[
            [pltpu.VMEM((K, c), jnp.float32)] * 2,
            [pltpu.VMEM((K,), jnp.int32)] * 3,
            [pltpu.VMEM((K,), jnp.int32)] * 3,
            [pltpu.VMEM((K,), jnp.float32)] * 3,
            pltpu.VMEM((ZR, c), jnp.float32),
            pltpu.VMEM_SHARED((n_pad, c), jnp.float32),
            [pltpu.SemaphoreType.DMA] * 2,
            [pltpu.SemaphoreType.DMA] * 2,
            [pltpu.SemaphoreType.DMA] * 3,
            [pltpu.SemaphoreType.DMA] * 3,
            pltpu.SemaphoreType.DMA,
        ],
    )
    def spmm(xf_h, colf_h, rowp_h, valp_h, out_h, rows, colv, rowv, valv,
             zbuf, acc, gsem, ssem, csem, rvsem, msem):
        cid = lax.axis_index("c")
        sid = lax.axis_index("s")
        tid = cid * NS + sid

        def zrow(i, carry):
            for j in range(c // 16):
                zbuf[i, pl.ds(j * 16, 16)] = jnp.zeros((16,), jnp.float32)
            return carry
        lax.fori_loop(0, ZR, zrow, 0)

        def col_start(b, g, m):
            pltpu.async_copy(colf_h.at[b, tid, g], colv[m], csem[m])

        def col_wait(m):
            pltpu.make_async_copy(colf_h.at[0, 0, 0], colv[m],
                                  csem[m]).wait()

        def rv_start(g, m):
            pltpu.async_copy(rowp_h.at[tid, g], rowv[m], rvsem[m])
            pltpu.async_copy(valp_h.at[tid, g], valv[m], rvsem[m])

        def rv_wait(m):
            pltpu.make_async_copy(rowp_h.at[0, 0], rowv[m], rvsem[m]).wait()
            pltpu.make_async_copy(valp_h.at[0, 0], valv[m], rvsem[m]).wait()

        def gather_start(m, r):
            pltpu.async_copy(xf_h.at[colv[m]], rows[r], gsem[r])

        def gather_wait(r):
            pltpu.make_async_copy(xf_h.at[colv[0]], rows[r],
                                  gsem[r]).wait()

        def scatter_start(m, r):
            pltpu.async_copy(rows[r], acc.at[rowv[m]], ssem[r], add=True)

        def scatter_wait(r):
            pltpu.make_async_copy(rows[r], acc.at[rowv[0]], ssem[r]).wait()

        def scale(m, r):
            @plsc.parallel_loop(0, K, unroll=2)
            def _(e):
                vs = plsc.load_gather(valv[m],
                                      [jnp.full((16,), e, jnp.int32)])
                for j in range(c // 16):
                    rows[r][e, pl.ds(j * 16, 16)] = (
                        rows[r][e, pl.ds(j * 16, 16)] * vs)

        for b in range(nb):
            zd = [pltpu.async_copy(
                      zbuf, acc.at[pl.ds(sid * rows_per_tile + z * ZR, ZR)],
                      msem)
                  for z in range(rows_per_tile // ZR)]
            col_start(b, 0, 0)
            col_start(b, 1, 1)
            rv_start(0, 0)
            rv_start(1, 1)
            col_wait(0)
            gather_start(0, 0)
            for d in zd:
                d.wait()
            plsc.subcore_barrier()

            def six(i6, carry):
                for u in range(UNROLL):
                    g = UNROLL * i6 + u
                    r = u % 2
                    m = u % 3
                    m1 = (u + 1) % 3
                    m2 = (u + 2) % 3
                    gather_wait(r)
                    if u < UNROLL - 2:
                        col_start(b, g + 2, m2)
                    else:
                        @pl.when(i6 < n_iter - 1)
                        def _():
                            col_start(b, g + 2, m2)
                    rv_wait(m)
                    scale(m, r)
                    scatter_start(m, r)
                    if u == 0:
                        @pl.when(i6 > 0)
                        def _():
                            scatter_wait(1)
                    else:
                        scatter_wait(1 - r)
                    if u < UNROLL - 2:
                        rv_start(g + 2, m2)
                    else:
                        @pl.when(i6 < n_iter - 1)
                        def _():
                            rv_start(g + 2, m2)
                    if u < UNROLL - 1:
                        col_wait(m1)
                        gather_start(m1, 1 - r)
                    else:
                        @pl.when(i6 < n_iter - 1)
                        def _():
                            col_wait(m1)
                            gather_start(m1, 1 - r)
                return carry
            lax.fori_loop(0, n_iter, six, 0)

            scatter_wait((steps - 1) % 2)
            plsc.subcore_barrier()

            pend = [None, None]
            for z in range(rows_per_tile // K):
                r = z % 2
                if pend[r] is not None:
                    pend[r].wait()
                r0 = sid * rows_per_tile + z * K
                pltpu.sync_copy(acc.at[pl.ds(r0, K)], rows[r])
                pend[r] = pltpu.async_copy(rows[r],
                                           out_h.at[cid, b, pl.ds(r0, K)],
                                           msem)
            for d in pend:
                if d is not None:
                    d.wait()

    return spmm(xf, colf, rowp, valp)


def _mlp_kernel(yp_ref, w1t_ref, b1_ref, w2t_ref, b2_ref, out_ref):
    y = yp_ref[0, 0] + yp_ref[1, 0]
    h = jnp.maximum(
        jnp.dot(y, w1t_ref[...], preferred_element_type=jnp.float32)
        + b1_ref[...], 0.0)
    out_ref[0] = (
        jnp.dot(h, w2t_ref[...], preferred_element_type=jnp.float32)
        + b2_ref[...])


def _mlp(ypart, w1t, b1r, w2t, b2r, *, nb, n_pad, c, c_out, blk=512):
    grid = (nb, n_pad // blk)
    return pl.pallas_call(
        _mlp_kernel,
        grid=grid,
        in_specs=[
            pl.BlockSpec((2, 1, blk, c), lambda b, n: (0, b, n, 0)),
            pl.BlockSpec((c, c), lambda b, n: (0, 0)),
            pl.BlockSpec((1, c), lambda b, n: (0, 0)),
            pl.BlockSpec((c, c_out), lambda b, n: (0, 0)),
            pl.BlockSpec((1, c_out), lambda b, n: (0, 0)),
        ],
        out_specs=pl.BlockSpec((1, blk, c_out), lambda b, n: (b, n, 0)),
        out_shape=jax.ShapeDtypeStruct((nb, n_pad, c_out), jnp.float32),
    )(ypart, w1t, b1r, w2t, b2r)


def kernel(x, A_idx, A_val, W1, b1, W2, b2):
    nb, n, c = x.shape
    e = A_val.shape[0]
    c_out = W2.shape[0]

    n_pad = ((n + NS * K - 1) // (NS * K)) * (NS * K)
    chunk = NW * K * UNROLL
    e_pad = ((e + chunk - 1) // chunk) * chunk
    steps = e_pad // (NW * K)

    row = A_idx[0].astype(jnp.int32)
    col = A_idx[1].astype(jnp.int32)
    pad = e_pad - e
    rowp = jnp.pad(row, (0, pad)).reshape(NW, steps, K)
    valp = jnp.pad(A_val, (0, pad)).reshape(NW, steps, K)
    colp = jnp.pad(col, (0, pad))
    colf = (colp[None, :] + (jnp.arange(nb, dtype=jnp.int32) * n)[:, None]
            ).reshape(nb, NW, steps, K)
    xf = x.reshape(nb * n, c)

    ypart = _sc_spmm(xf, colf, rowp, valp, n_pad=n_pad, steps=steps, nb=nb,
                     c=c)

    out = _mlp(ypart, W1.T, b1.reshape(1, -1), W2.T, b2.reshape(1, -1),
               nb=nb, n_pad=n_pad, c=c, c_out=c_out)
    return out[:, :n, :]


# speedup vs baseline: 1.0059x; 1.0059x over previous
import functools

import jax
import jax.numpy as jnp
from jax import lax
from jax.experimental import pallas as pl
from jax.experimental.pallas import tpu as pltpu
from jax.experimental.pallas import tpu_sc as plsc

NC = 2
NS = 16
NW = NC * NS
K = 128
ZR = 32
UNROLL = 6


def _sc_spmm(xf, colf, rowp, valp, *, n_pad, steps, nb, c):
    rows_per_tile = n_pad // NS
    n_iter = steps // UNROLL

    mesh = plsc.VectorSubcoreMesh(core_axis_name="c", subcore_axis_name="s")

    @functools.partial(
        pl.kernel,
        out_type=jax.ShapeDtypeStruct((NC, nb, n_pad, c), jnp.float32),
        mesh=mesh,
        compiler_params=pltpu.CompilerParams(needs_layout_passes=False),
        scratch_types=[
            [pltpu.VMEM((K, c), jnp.float32)] * 2,
            [pltpu.VMEM((K,), jnp.int32)] * 3,
            [pltpu.VMEM((K,), jnp.int32)] * 3,
            [pltpu.VMEM((K,), jnp.float32)] * 3,
            pltpu.VMEM((ZR, c), jnp.float32),
            pltpu.VMEM_SHARED((n_pad, c), jnp.float32),
            [pltpu.SemaphoreType.DMA] * 2,
            [pltpu.SemaphoreType.DMA] * 2,
            [pltpu.SemaphoreType.DMA] * 3,
            [pltpu.SemaphoreType.DMA] * 3,
            pltpu.SemaphoreType.DMA,
        ],
    )
    def spmm(xf_h, colf_h, rowp_h, valp_h, out_h, rows, colv, rowv, valv,
             zbuf, acc, gsem, ssem, csem, rvsem, msem):
        cid = lax.axis_index("c")
        sid = lax.axis_index("s")
        tid = cid * NS + sid

        def zrow(i, carry):
            for j in range(c // 16):
                zbuf[i, pl.ds(j * 16, 16)] = jnp.zeros((16,), jnp.float32)
            return carry
        lax.fori_loop(0, ZR, zrow, 0)

        def col_start(b, g, m):
            pltpu.async_copy(colf_h.at[b, tid, g], colv[m], csem[m])

        def col_wait(m):
            pltpu.make_async_copy(colf_h.at[0, 0, 0], colv[m],
                                  csem[m]).wait()

        def rv_start(g, m):
            pltpu.async_copy(rowp_h.at[tid, g], rowv[m], rvsem[m])
            pltpu.async_copy(valp_h.at[tid, g], valv[m], rvsem[m])

        def rv_wait(m):
            pltpu.make_async_copy(rowp_h.at[0, 0], rowv[m], rvsem[m]).wait()
            pltpu.make_async_copy(valp_h.at[0, 0], valv[m], rvsem[m]).wait()

        def gather_start(m, r):
            pltpu.async_copy(xf_h.at[colv[m]], rows[r], gsem[r])

        def gather_wait(r):
            pltpu.make_async_copy(xf_h.at[colv[0]], rows[r],
                                  gsem[r]).wait()

        def scatter_start(m, r):
            pltpu.async_copy(rows[r], acc.at[rowv[m]], ssem[r], add=True)

        def scatter_wait(r):
            pltpu.make_async_copy(rows[r], acc.at[rowv[0]], ssem[r]).wait()

        def scale(m, r):
            def mul(e, carry):
                vs = plsc.load_gather(valv[m],
                                      [jnp.full((16,), e, jnp.int32)])
                for j in range(c // 16):
                    rows[r][e, pl.ds(j * 16, 16)] = (
                        rows[r][e, pl.ds(j * 16, 16)] * vs)
                return carry
            lax.fori_loop(0, K, mul, 0)

        for b in range(nb):
            zd = [pltpu.async_copy(
                      zbuf, acc.at[pl.ds(sid * rows_per_tile + z * ZR, ZR)],
                      msem)
                  for z in range(rows_per_tile // ZR)]
            col_start(b, 0, 0)
            col_start(b, 1, 1)
            rv_start(0, 0)
            rv_start(1, 1)
            col_wait(0)
            gather_start(0, 0)
            for d in zd:
                d.wait()
            plsc.subcore_barrier()

            def six(i6, carry):
                for u in range(UNROLL):
                    g = UNROLL * i6 + u
                    r = u % 2
                    m = u % 3
                    m1 = (u + 1) % 3
                    m2 = (u + 2) % 3
                    gather_wait(r)
                    if u < UNROLL - 2:
                        col_start(b, g + 2, m2)
                    else:
                        @pl.when(i6 < n_iter - 1)
                        def _():
                            col_start(b, g + 2, m2)
                    rv_wait(m)
                    scale(m, r)
                    scatter_start(m, r)
                    if u == 0:
                        @pl.when(i6 > 0)
                        def _():
                            scatter_wait(1)
                    else:
                        scatter_wait(1 - r)
                    if u < UNROLL - 2:
                        rv_start(g + 2, m2)
                    else:
                        @pl.when(i6 < n_iter - 1)
                        def _():
                            rv_start(g + 2, m2)
                    if u < UNROLL - 1:
                        col_wait(m1)
                        gather_start(m1, 1 - r)
                    else:
                        @pl.when(i6 < n_iter - 1)
                        def _():
                            col_wait(m1)
                            gather_start(m1, 1 - r)
                return carry
            lax.fori_loop(0, n_iter, six, 0)

            scatter_wait((steps - 1) % 2)
            plsc.subcore_barrier()

            pend = [None, None]
            for z in range(rows_per_tile // K):
                r = z % 2
                if pend[r] is not None:
                    pend[r].wait()
                r0 = sid * rows_per_tile + z * K
                pltpu.sync_copy(acc.at[pl.ds(r0, K)], rows[r])
                pend[r] = pltpu.async_copy(rows[r],
                                           out_h.at[cid, b, pl.ds(r0, K)],
                                           msem)
            for d in pend:
                if d is not None:
                    d.wait()

    return spmm(xf, colf, rowp, valp)


def _mlp_kernel(yp_ref, w1t_ref, b1_ref, w2t_ref, b2_ref, out_ref):
    y = yp_ref[0, 0] + yp_ref[1, 0]
    h = jnp.maximum(
        jnp.dot(y, w1t_ref[...], preferred_element_type=jnp.float32)
        + b1_ref[...], 0.0)
    out_ref[0] = (
        jnp.dot(h, w2t_ref[...], preferred_element_type=jnp.float32)
        + b2_ref[...])


def _mlp(ypart, w1t, b1r, w2t, b2r, *, nb, n_pad, c, c_out, blk=512):
    grid = (nb, n_pad // blk)
    return pl.pallas_call(
        _mlp_kernel,
        grid=grid,
        in_specs=[
            pl.BlockSpec((2, 1, blk, c), lambda b, n: (0, b, n, 0)),
            pl.BlockSpec((c, c), lambda b, n: (0, 0)),
            pl.BlockSpec((1, c), lambda b, n: (0, 0)),
            pl.BlockSpec((c, c_out), lambda b, n: (0, 0)),
            pl.BlockSpec((1, c_out), lambda b, n: (0, 0)),
        ],
        out_specs=pl.BlockSpec((1, blk, c_out), lambda b, n: (b, n, 0)),
        out_shape=jax.ShapeDtypeStruct((nb, n_pad, c_out), jnp.float32),
    )(ypart, w1t, b1r, w2t, b2r)


def kernel(x, A_idx, A_val, W1, b1, W2, b2):
    nb, n, c = x.shape
    e = A_val.shape[0]
    c_out = W2.shape[0]

    n_pad = ((n + NS * K - 1) // (NS * K)) * (NS * K)
    chunk = NW * K * UNROLL
    e_pad = ((e + chunk - 1) // chunk) * chunk
    steps = e_pad // (NW * K)

    row = A_idx[0].astype(jnp.int32)
    col = A_idx[1].astype(jnp.int32)
    pad = e_pad - e
    rowp = jnp.pad(row, (0, pad)).reshape(NW, steps, K)
    valp = jnp.pad(A_val, (0, pad)).reshape(NW, steps, K)
    colp = jnp.pad(col, (0, pad))
    colf = (colp[None, :] + (jnp.arange(nb, dtype=jnp.int32) * n)[:, None]
            ).reshape(nb, NW, steps, K)
    xf = x.reshape(nb * n, c)

    ypart = _sc_spmm(xf, colf, rowp, valp, n_pad=n_pad, steps=steps, nb=nb,
                     c=c)

    out = _mlp(ypart, W1.T, b1.reshape(1, -1), W2.T, b2.reshape(1, -1),
               nb=nb, n_pad=n_pad, c=c, c_out=c_out)
    return out[:, :n, :]


# speedup vs baseline: 1.0095x; 1.0036x over previous
import functools

import jax
import jax.numpy as jnp
from jax import lax
from jax.experimental import pallas as pl
from jax.experimental.pallas import tpu as pltpu
from jax.experimental.pallas import tpu_sc as plsc

NC = 2
NS = 16
NW = NC * NS
K = 128
ZR = 32
UNROLL = 6


def _sc_spmm(xf, colf, rowp, valp, *, n_pad, steps, nb, c):
    rows_per_tile = n_pad // NS
    n_iter = steps // UNROLL

    mesh = plsc.VectorSubcoreMesh(core_axis_name="c", subcore_axis_name="s")

    @functools.partial(
        pl.kernel,
        out_type=jax.ShapeDtypeStruct((NC, nb, n_pad, c), jnp.float32),
        mesh=mesh,
        compiler_params=pltpu.CompilerParams(needs_layout_passes=False),
        scratch_types=[
            [pltpu.VMEM((K, c), jnp.float32)] * 2,
            [pltpu.VMEM((K,), jnp.int32)] * 3,
            [pltpu.VMEM((K,), jnp.int32)] * 3,
            [pltpu.VMEM((K,), jnp.float32)] * 3,
            pltpu.VMEM((ZR, c), jnp.float32),
            pltpu.VMEM_SHARED((n_pad, c), jnp.float32),
            [pltpu.SemaphoreType.DMA] * 2,
            [pltpu.SemaphoreType.DMA] * 2,
            [pltpu.SemaphoreType.DMA] * 3,
            [pltpu.SemaphoreType.DMA] * 3,
            pltpu.SemaphoreType.DMA,
        ],
    )
    def spmm(xf_h, colf_h, rowp_h, valp_h, out_h, rows, colv, rowv, valv,
             zbuf, acc, gsem, ssem, csem, rvsem, msem):
        cid = lax.axis_index("c")
        sid = lax.axis_index("s")
        tid = cid * NS + sid

        def zrow(i, carry):
            for j in range(c // 16):
                zbuf[i, pl.ds(j * 16, 16)] = jnp.zeros((16,), jnp.float32)
            return carry
        lax.fori_loop(0, ZR, zrow, 0)

        def col_start(b, g, m):
            pltpu.async_copy(colf_h.at[b, tid, g], colv[m], csem[m])

        def col_wait(m):
            pltpu.make_async_copy(colf_h.at[0, 0, 0], colv[m],
                                  csem[m]).wait()

        def rv_start(g, m):
            pltpu.async_copy(rowp_h.at[tid, g], rowv[m], rvsem[m])
            pltpu.async_copy(valp_h.at[tid, g], valv[m], rvsem[m])

        def rv_wait(m):
            pltpu.make_async_copy(rowp_h.at[0, 0], rowv[m], rvsem[m]).wait()
            pltpu.make_async_copy(valp_h.at[0, 0], valv[m], rvsem[m]).wait()

        def gather_start(m, r):
            pltpu.async_copy(xf_h.at[colv[m]], rows[r], gsem[r])

        def gather_wait(r):
            pltpu.make_async_copy(xf_h.at[colv[0]], rows[r],
                                  gsem[r]).wait()

        def scatter_start(m, r):
            pltpu.async_copy(rows[r], acc.at[rowv[m]], ssem[r], add=True)

        def scatter_wait(r):
            pltpu.make_async_copy(rows[r], acc.at[rowv[0]], ssem[r]).wait()

        def scale(m, r):
            def mul(e, carry):
                vs = plsc.load_gather(valv[m],
                                      [jnp.full((16,), e, jnp.int32)])
                for j in range(c // 16):
                    rows[r][e, pl.ds(j * 16, 16)] = (
                        rows[r][e, pl.ds(j * 16, 16)] * vs)
                return carry
            lax.fori_loop(0, K, mul, 0)

        for b in range(nb):
            zd = [pltpu.async_copy(
                      zbuf, acc.at[pl.ds(sid * rows_per_tile + z * ZR, ZR)],
                      msem)
                  for z in range(rows_per_tile // ZR)]
            col_start(b, 0, 0)
            col_start(b, 1, 1)
            rv_start(0, 0)
            rv_start(1, 1)
            col_wait(0)
            gather_start(0, 0)
            for d in zd:
                d.wait()
            plsc.subcore_barrier()

            def six(i6, carry):
                for u in range(UNROLL):
                    g = UNROLL * i6 + u
                    r = u % 2
                    m = u % 3
                    m1 = (u + 1) % 3
                    m2 = (u + 2) % 3
                    gather_wait(r)
                    if u < UNROLL - 2:
                        col_start(b, g + 2, m2)
                    else:
                        @pl.when(i6 < n_iter - 1)
                        def _():
                            col_start(b, g + 2, m2)
                    rv_wait(m)
                    scale(m, r)
                    pltpu.sync_copy(rows[r], acc.at[rowv[m]], add=True)
                    if u < UNROLL - 2:
                        rv_start(g + 2, m2)
                    else:
                        @pl.when(i6 < n_iter - 1)
                        def _():
                            rv_start(g + 2, m2)
                    if u < UNROLL - 1:
                        col_wait(m1)
                        gather_start(m1, 1 - r)
                    else:
                        @pl.when(i6 < n_iter - 1)
                        def _():
                            col_wait(m1)
                            gather_start(m1, 1 - r)
                return carry
            lax.fori_loop(0, n_iter, six, 0)
            plsc.subcore_barrier()

            pend = [None, None]
            for z in range(rows_per_tile // K):
                r = z % 2
                if pend[r] is not None:
                    pend[r].wait()
                r0 = sid * rows_per_tile + z * K
                pltpu.sync_copy(acc.at[pl.ds(r0, K)], rows[r])
                pend[r] = pltpu.async_copy(rows[r],
                                           out_h.at[cid, b, pl.ds(r0, K)],
                                           msem)
            for d in pend:
                if d is not None:
                    d.wait()

    return spmm(xf, colf, rowp, valp)


def _mlp_kernel(yp_ref, w1t_ref, b1_ref, w2t_ref, b2_ref, out_ref):
    y = yp_ref[0, 0] + yp_ref[1, 0]
    h = jnp.maximum(
        jnp.dot(y, w1t_ref[...], preferred_element_type=jnp.float32)
        + b1_ref[...], 0.0)
    out_ref[0] = (
        jnp.dot(h, w2t_ref[...], preferred_element_type=jnp.float32)
        + b2_ref[...])


def _mlp(ypart, w1t, b1r, w2t, b2r, *, nb, n_pad, c, c_out, blk=512):
    grid = (nb, n_pad // blk)
    return pl.pallas_call(
        _mlp_kernel,
        grid=grid,
        in_specs=[
            pl.BlockSpec((2, 1, blk, c), lambda b, n: (0, b, n, 0)),
            pl.BlockSpec((c, c), lambda b, n: (0, 0)),
            pl.BlockSpec((1, c), lambda b, n: (0, 0)),
            pl.BlockSpec((c, c_out), lambda b, n: (0, 0)),
            pl.BlockSpec((1, c_out), lambda b, n: (0, 0)),
        ],
        out_specs=pl.BlockSpec((1, blk, c_out), lambda b, n: (b, n, 0)),
        out_shape=jax.ShapeDtypeStruct((nb, n_pad, c_out), jnp.float32),
    )(ypart, w1t, b1r, w2t, b2r)


def kernel(x, A_idx, A_val, W1, b1, W2, b2):
    nb, n, c = x.shape
    e = A_val.shape[0]
    c_out = W2.shape[0]

    n_pad = ((n + NS * K - 1) // (NS * K)) * (NS * K)
    chunk = NW * K * UNROLL
    e_pad = ((e + chunk - 1) // chunk) * chunk
    steps = e_pad // (NW * K)

    row = A_idx[0].astype(jnp.int32)
    col = A_idx[1].astype(jnp.int32)
    pad = e_pad - e
    rowp = jnp.pad(row, (0, pad)).reshape(NW, steps, K)
    valp = jnp.pad(A_val, (0, pad)).reshape(NW, steps, K)
    colp = jnp.pad(col, (0, pad))
    colf = (colp[None, :] + (jnp.arange(nb, dtype=jnp.int32) * n)[:, None]
            ).reshape(nb, NW, steps, K)
    xf = x.reshape(nb * n, c)

    ypart = _sc_spmm(xf, colf, rowp, valp, n_pad=n_pad, steps=steps, nb=nb,
                     c=c)

    out = _mlp(ypart, W1.T, b1.reshape(1, -1), W2.T, b2.reshape(1, -1),
               nb=nb, n_pad=n_pad, c=c, c_out=c_out)
    return out[:, :n, :]


# speedup vs baseline: 1.8782x; 1.8604x over previous
import functools

import jax
import jax.numpy as jnp
from jax import lax
from jax.experimental import pallas as pl
from jax.experimental.pallas import tpu as pltpu
from jax.experimental.pallas import tpu_sc as plsc

NC = 2
NS = 16
NW = NC * NS
K = 128
ZR = 32
UNROLL = 2


def _sc_spmm(xf, colf, rowp, valp, *, n_pad, steps, nb, c):
    rows_per_tile = n_pad // NS
    n_iter = steps // UNROLL

    mesh = plsc.VectorSubcoreMesh(core_axis_name="c", subcore_axis_name="s")

    @functools.partial(
        pl.kernel,
        out_type=jax.ShapeDtypeStruct((NC, nb, n_pad, c), jnp.float32),
        mesh=mesh,
        compiler_params=pltpu.CompilerParams(needs_layout_passes=False),
        scratch_types=[
            [pltpu.VMEM((K, c), jnp.float32)] * 2,
            [pltpu.VMEM((K,), jnp.int32)] * 2,
            [pltpu.VMEM((K,), jnp.int32)] * 2,
            [pltpu.VMEM((K,), jnp.float32)] * 2,
            pltpu.VMEM((ZR, c), jnp.float32),
            pltpu.VMEM_SHARED((n_pad, c), jnp.float32),
            [pltpu.SemaphoreType.DMA] * 2,
            [pltpu.SemaphoreType.DMA] * 2,
            [pltpu.SemaphoreType.DMA] * 2,
            pltpu.SemaphoreType.DMA,
        ],
    )
    def spmm(xf_h, colf_h, rowp_h, valp_h, out_h, rows, colv, rowv, valv,
             zbuf, acc, gsem, csem, rvsem, msem):
        cid = lax.axis_index("c")
        sid = lax.axis_index("s")
        tid = cid * NS + sid

        def zrow(i, carry):
            for j in range(c // 16):
                zbuf[i, pl.ds(j * 16, 16)] = jnp.zeros((16,), jnp.float32)
            return carry
        lax.fori_loop(0, ZR, zrow, 0)

        def col_start(b, g, m):
            pltpu.async_copy(colf_h.at[b, tid, g], colv[m], csem[m])

        def col_wait(m):
            pltpu.make_async_copy(colf_h.at[0, 0, 0], colv[m],
                                  csem[m]).wait()

        def rv_start(g, m):
            pltpu.async_copy(rowp_h.at[tid, g], rowv[m], rvsem[m])
            pltpu.async_copy(valp_h.at[tid, g], valv[m], rvsem[m])

        def rv_wait(m):
            pltpu.make_async_copy(rowp_h.at[0, 0], rowv[m], rvsem[m]).wait()
            pltpu.make_async_copy(valp_h.at[0, 0], valv[m], rvsem[m]).wait()

        def gather_start(m, r):
            pltpu.async_copy(xf_h.at[colv[m]], rows[r], gsem[r])

        def gather_wait(r):
            pltpu.make_async_copy(xf_h.at[colv[0]], rows[r],
                                  gsem[r]).wait()

        def scale(m, r):
            def mul(e, carry):
                vs = plsc.load_gather(valv[m],
                                      [jnp.full((16,), e, jnp.int32)])
                for j in range(c // 16):
                    rows[r][e, pl.ds(j * 16, 16)] = (
                        rows[r][e, pl.ds(j * 16, 16)] * vs)
                return carry
            lax.fori_loop(0, K, mul, 0)

        for b in range(nb):
            zd = [pltpu.async_copy(
                      zbuf, acc.at[pl.ds(sid * rows_per_tile + z * ZR, ZR)],
                      msem)
                  for z in range(rows_per_tile // ZR)]
            col_start(b, 0, 0)
            rv_start(0, 0)
            col_wait(0)
            rv_wait(0)
            gather_start(0, 0)
            for d in zd:
                d.wait()
            plsc.subcore_barrier()

            def one(g, r):
                col_start(b, g + 1, 1 - r)
                rv_start(g + 1, 1 - r)
                gather_wait(r)
                scale(r, r)
                pltpu.sync_copy(rows[r], acc.at[rowv[r]], add=True)
                col_wait(1 - r)
                rv_wait(1 - r)
                gather_start(1 - r, 1 - r)

            def pair(i2, carry):
                one(2 * i2, 0)
                one(2 * i2 + 1, 1)
                return carry
            lax.fori_loop(0, (steps - 2) // 2, pair, 0)
            one(steps - 2, 0)
            gather_wait(1)
            scale(1, 1)
            pltpu.sync_copy(rows[1], acc.at[rowv[1]], add=True)
            plsc.subcore_barrier()

            pend = [None, None]
            for z in range(rows_per_tile // K):
                r = z % 2
                if pend[r] is not None:
                    pend[r].wait()
                r0 = sid * rows_per_tile + z * K
                pltpu.sync_copy(acc.at[pl.ds(r0, K)], rows[r])
                pend[r] = pltpu.async_copy(rows[r],
                                           out_h.at[cid, b, pl.ds(r0, K)],
                                           msem)
            for d in pend:
                if d is not None:
                    d.wait()

    return spmm(xf, colf, rowp, valp)


def _mlp_kernel(yp_ref, w1t_ref, b1_ref, w2t_ref, b2_ref, out_ref):
    y = yp_ref[0, 0] + yp_ref[1, 0]
    h = jnp.maximum(
        jnp.dot(y, w1t_ref[...], preferred_element_type=jnp.float32)
        + b1_ref[...], 0.0)
    out_ref[0] = (
        jnp.dot(h, w2t_ref[...], preferred_element_type=jnp.float32)
        + b2_ref[...])


def _mlp(ypart, w1t, b1r, w2t, b2r, *, nb, n_pad, c, c_out, blk=512):
    grid = (nb, n_pad // blk)
    return pl.pallas_call(
        _mlp_kernel,
        grid=grid,
        in_specs=[
            pl.BlockSpec((2, 1, blk, c), lambda b, n: (0, b, n, 0)),
            pl.BlockSpec((c, c), lambda b, n: (0, 0)),
            pl.BlockSpec((1, c), lambda b, n: (0, 0)),
            pl.BlockSpec((c, c_out), lambda b, n: (0, 0)),
            pl.BlockSpec((1, c_out), lambda b, n: (0, 0)),
        ],
        out_specs=pl.BlockSpec((1, blk, c_out), lambda b, n: (b, n, 0)),
        out_shape=jax.ShapeDtypeStruct((nb, n_pad, c_out), jnp.float32),
    )(ypart, w1t, b1r, w2t, b2r)


def kernel(x, A_idx, A_val, W1, b1, W2, b2):
    nb, n, c = x.shape
    e = A_val.shape[0]
    c_out = W2.shape[0]

    n_pad = ((n + NS * K - 1) // (NS * K)) * (NS * K)
    chunk = NW * K * UNROLL
    e_pad = ((e + chunk - 1) // chunk) * chunk
    steps = e_pad // (NW * K)

    row = A_idx[0].astype(jnp.int32)
    col = A_idx[1].astype(jnp.int32)
    pad = e_pad - e
    rowp = jnp.pad(row, (0, pad)).reshape(NW, steps, K)
    valp = jnp.pad(A_val, (0, pad)).reshape(NW, steps, K)
    colp = jnp.pad(col, (0, pad))
    colf = (colp[None, :] + (jnp.arange(nb, dtype=jnp.int32) * n)[:, None]
            ).reshape(nb, NW, steps, K)
    xf = x.reshape(nb * n, c)

    ypart = _sc_spmm(xf, colf, rowp, valp, n_pad=n_pad, steps=steps, nb=nb,
                     c=c)

    out = _mlp(ypart, W1.T, b1.reshape(1, -1), W2.T, b2.reshape(1, -1),
               nb=nb, n_pad=n_pad, c=c, c_out=c_out)
    return out[:, :n, :]


# speedup vs baseline: 2.5668x; 1.3667x over previous
import functools

import jax
import jax.numpy as jnp
from jax import lax
from jax.experimental import pallas as pl
from jax.experimental.pallas import tpu as pltpu
from jax.experimental.pallas import tpu_sc as plsc

NC = 2
NS = 16
NW = NC * NS
K = 64
D = 4
ZR = 32


def _sc_spmm(xf, colf, rowp, valp, *, n_pad, steps, nb, c):
    rows_per_tile = n_pad // NS
    n_iter = steps // D

    mesh = plsc.VectorSubcoreMesh(core_axis_name="c", subcore_axis_name="s")

    @functools.partial(
        pl.kernel,
        out_type=jax.ShapeDtypeStruct((NC, nb, n_pad, c), jnp.float32),
        mesh=mesh,
        compiler_params=pltpu.CompilerParams(needs_layout_passes=False),
        scratch_types=[
            [pltpu.VMEM((K, c), jnp.float32)] * D,
            [pltpu.VMEM((K,), jnp.int32)] * D,
            [pltpu.VMEM((K,), jnp.int32)] * D,
            [pltpu.VMEM((K,), jnp.float32)] * D,
            pltpu.VMEM((ZR, c), jnp.float32),
            pltpu.VMEM_SHARED((n_pad, c), jnp.float32),
            [pltpu.SemaphoreType.DMA] * D,
            [pltpu.SemaphoreType.DMA] * D,
            [pltpu.SemaphoreType.DMA] * D,
            [pltpu.SemaphoreType.DMA] * D,
            pltpu.SemaphoreType.DMA,
        ],
    )
    def spmm(xf_h, colf_h, rowp_h, valp_h, out_h, rows, colv, rowv, valv,
             zbuf, acc, gsem, ssem, csem, rvsem, msem):
        cid = lax.axis_index("c")
        sid = lax.axis_index("s")
        tid = cid * NS + sid

        def zrow(i, carry):
            for j in range(c // 16):
                zbuf[i, pl.ds(j * 16, 16)] = jnp.zeros((16,), jnp.float32)
            return carry
        lax.fori_loop(0, ZR, zrow, 0)

        def col_start(b, g, m):
            pltpu.async_copy(colf_h.at[b, tid, g], colv[m], csem[m])

        def col_wait(m):
            pltpu.make_async_copy(colf_h.at[0, 0, 0], colv[m],
                                  csem[m]).wait()

        def rv_start(g, m):
            pltpu.async_copy(rowp_h.at[tid, g], rowv[m], rvsem[m])
            pltpu.async_copy(valp_h.at[tid, g], valv[m], rvsem[m])

        def rv_wait(m):
            pltpu.make_async_copy(rowp_h.at[0, 0], rowv[m], rvsem[m]).wait()
            pltpu.make_async_copy(valp_h.at[0, 0], valv[m], rvsem[m]).wait()

        def gather_start(m):
            pltpu.async_copy(xf_h.at[colv[m]], rows[m], gsem[m])

        def gather_wait(m):
            pltpu.make_async_copy(xf_h.at[colv[0]], rows[m],
                                  gsem[m]).wait()

        def scatter_start(m):
            pltpu.async_copy(rows[m], acc.at[rowv[m]], ssem[m], add=True)

        def scatter_wait(m):
            pltpu.make_async_copy(rows[m], acc.at[rowv[0]], ssem[m]).wait()

        def scale(m):
            def mul(e, carry):
                vs = plsc.load_gather(valv[m],
                                      [jnp.full((16,), e, jnp.int32)])
                for j in range(c // 16):
                    rows[m][e, pl.ds(j * 16, 16)] = (
                        rows[m][e, pl.ds(j * 16, 16)] * vs)
                return carry
            lax.fori_loop(0, K, mul, 0)

        def one(b, g, u, *, first=False, last=False):
            r = u % D
            r2 = (u + 2) % D
            r3 = (u + 3) % D
            live2 = not (last and u >= D - 2)
            live3 = not (last and u >= D - 3)
            if not (first and u < 2):
                scatter_wait(r2)
            if live2:
                col_wait(r2)
                gather_start(r2)
                rv_start(g + 2, r2)
            if live3:
                col_start(b, g + 3, r3)
            rv_wait(r)
            gather_wait(r)
            scale(r)
            scatter_start(r)

        for b in range(nb):
            zd = [pltpu.async_copy(
                      zbuf, acc.at[pl.ds(sid * rows_per_tile + z * ZR, ZR)],
                      msem)
                  for z in range(rows_per_tile // ZR)]
            for m in range(3):
                col_start(b, m, m)
            rv_start(0, 0)
            rv_start(1, 1)
            col_wait(0)
            gather_start(0)
            col_wait(1)
            gather_start(1)
            for d in zd:
                d.wait()
            plsc.subcore_barrier()

            for u in range(D):
                one(b, u, u, first=True)

            def ring(i4, carry):
                for u in range(D):
                    one(b, D * i4 + u, u)
                return carry
            lax.fori_loop(1, n_iter - 1, ring, 0)

            for u in range(D):
                one(b, steps - D + u, u, last=True)
            scatter_wait((steps - 2) % D)
            scatter_wait((steps - 1) % D)
            plsc.subcore_barrier()

            pend = [None] * D
            for z in range(rows_per_tile // K):
                r = z % D
                if pend[r] is not None:
                    pend[r].wait()
                r0 = sid * rows_per_tile + z * K
                pltpu.sync_copy(acc.at[pl.ds(r0, K)], rows[r])
                pend[r] = pltpu.async_copy(rows[r],
                                           out_h.at[cid, b, pl.ds(r0, K)],
                                           msem)
            for d in pend:
                if d is not None:
                    d.wait()

    return spmm(xf, colf, rowp, valp)


def _mlp_kernel(yp_ref, w1t_ref, b1_ref, w2t_ref, b2_ref, out_ref):
    y = yp_ref[0, 0] + yp_ref[1, 0]
    h = jnp.maximum(
        jnp.dot(y, w1t_ref[...], preferred_element_type=jnp.float32)
        + b1_ref[...], 0.0)
    out_ref[0] = (
        jnp.dot(h, w2t_ref[...], preferred_element_type=jnp.float32)
        + b2_ref[...])


def _mlp(ypart, w1t, b1r, w2t, b2r, *, nb, n_pad, c, c_out, blk=512):
    grid = (nb, n_pad // blk)
    return pl.pallas_call(
        _mlp_kernel,
        grid=grid,
        in_specs=[
            pl.BlockSpec((2, 1, blk, c), lambda b, n: (0, b, n, 0)),
            pl.BlockSpec((c, c), lambda b, n: (0, 0)),
            pl.BlockSpec((1, c), lambda b, n: (0, 0)),
            pl.BlockSpec((c, c_out), lambda b, n: (0, 0)),
            pl.BlockSpec((1, c_out), lambda b, n: (0, 0)),
        ],
        out_specs=pl.BlockSpec((1, blk, c_out), lambda b, n: (b, n, 0)),
        out_shape=jax.ShapeDtypeStruct((nb, n_pad, c_out), jnp.float32),
    )(ypart, w1t, b1r, w2t, b2r)


def kernel(x, A_idx, A_val, W1, b1, W2, b2):
    nb, n, c = x.shape
    e = A_val.shape[0]
    c_out = W2.shape[0]

    n_pad = ((n + NS * 128 - 1) // (NS * 128)) * (NS * 128)
    chunk = NW * K * D
    e_pad = ((e + chunk - 1) // chunk) * chunk
    steps = e_pad // (NW * K)

    row = A_idx[0].astype(jnp.int32)
    col = A_idx[1].astype(jnp.int32)
    pad = e_pad - e
    rowp = jnp.pad(row, (0, pad)).reshape(NW, steps, K)
    valp = jnp.pad(A_val, (0, pad)).reshape(NW, steps, K)
    colp = jnp.pad(col, (0, pad))
    colf = (colp[None, :] + (jnp.arange(nb, dtype=jnp.int32) * n)[:, None]
            ).reshape(nb, NW, steps, K)
    xf = x.reshape(nb * n, c)

    ypart = _sc_spmm(xf, colf, rowp, valp, n_pad=n_pad, steps=steps, nb=nb,
                     c=c)

    out = _mlp(ypart, W1.T, b1.reshape(1, -1), W2.T, b2.reshape(1, -1),
               nb=nb, n_pad=n_pad, c=c, c_out=c_out)
    return out[:, :n, :]


# speedup vs baseline: 8.0764x; 3.1464x over previous
import functools

import jax
import jax.numpy as jnp
from jax import lax
from jax.experimental import pallas as pl
from jax.experimental.pallas import tpu as pltpu
from jax.experimental.pallas import tpu_sc as plsc

NC = 2
NS = 16
NW = NC * NS
K = 64
D = 4
ZR = 32


def _sc_spmm(xf, colf, rowp, valp, *, n_pad, steps, nb, c):
    rows_per_tile = n_pad // NS
    n_iter = steps // D

    mesh = plsc.VectorSubcoreMesh(core_axis_name="c", subcore_axis_name="s")

    @functools.partial(
        pl.kernel,
        out_type=jax.ShapeDtypeStruct((NC, nb, n_pad, c), jnp.float32),
        mesh=mesh,
        compiler_params=pltpu.CompilerParams(needs_layout_passes=False),
        scratch_types=[
            [pltpu.VMEM((K, c), jnp.float32)] * D,
            [pltpu.VMEM((K,), jnp.int32)] * D,
            [pltpu.VMEM((K,), jnp.int32)] * D,
            [pltpu.VMEM((K,), jnp.float32)] * D,
            pltpu.VMEM((ZR, c), jnp.float32),
            pltpu.VMEM_SHARED((n_pad, c), jnp.float32),
            [pltpu.SemaphoreType.DMA] * D,
            [pltpu.SemaphoreType.DMA] * D,
            [pltpu.SemaphoreType.DMA] * D,
            [pltpu.SemaphoreType.DMA] * D,
            pltpu.SemaphoreType.DMA,
        ],
    )
    def spmm(xf_h, colf_h, rowp_h, valp_h, out_h, rows, colv, rowv, valv,
             zbuf, acc, gsem, ssem, csem, rvsem, msem):
        cid = lax.axis_index("c")
        sid = lax.axis_index("s")
        tid = cid * NS + sid

        def zrow(i, carry):
            for j in range(c // 16):
                zbuf[i, pl.ds(j * 16, 16)] = jnp.zeros((16,), jnp.float32)
            return carry
        lax.fori_loop(0, ZR, zrow, 0)

        def col_start(b, g, m):
            pltpu.async_copy(colf_h.at[b, tid, g], colv[m], csem[m])

        def col_wait(m):
            pltpu.make_async_copy(colf_h.at[0, 0, 0], colv[m],
                                  csem[m]).wait()

        def rv_start(g, m):
            pltpu.async_copy(rowp_h.at[tid, g], rowv[m], rvsem[m])
            pltpu.async_copy(valp_h.at[tid, g], valv[m], rvsem[m])

        def rv_wait(m):
            pltpu.make_async_copy(rowp_h.at[0, 0], rowv[m], rvsem[m]).wait()
            pltpu.make_async_copy(valp_h.at[0, 0], valv[m], rvsem[m]).wait()

        def gather_start(m):
            pltpu.async_copy(xf_h.at[colv[m]], rows[m], gsem[m])

        def gather_wait(m):
            pltpu.make_async_copy(xf_h.at[colv[0]], rows[m],
                                  gsem[m]).wait()

        def scatter_start(m):
            pltpu.async_copy(rows[m], acc.at[rowv[m]], ssem[m], add=True)

        def scatter_wait(m):
            pltpu.make_async_copy(rows[m], acc.at[rowv[0]], ssem[m]).wait()

        def scale(m):
            def mul(e, carry):
                vs = plsc.load_gather(valv[m],
                                      [jnp.full((16,), e, jnp.int32)])
                for j in range(c // 16):
                    rows[m][e, pl.ds(j * 16, 16)] = (
                        rows[m][e, pl.ds(j * 16, 16)] * vs)
                return carry
            lax.fori_loop(0, K, mul, 0)

        def one(b, g, u, *, first=False, last=False):
            r = u % D
            r2 = (u + 2) % D
            r3 = (u + 3) % D
            live2 = not (last and u >= D - 2)
            live3 = not (last and u >= D - 3)
            if not (first and u < 2):
                scatter_wait(r2)
            if live2:
                col_wait(r2)
                gather_start(r2)
                rv_start(g + 2, r2)
            if live3:
                col_start(b, g + 3, r3)
            rv_wait(r)
            gather_wait(r)
            scale(r)
            scatter_start(r)

        for b in range(nb):
            zd = [pltpu.async_copy(
                      zbuf, acc.at[pl.ds(sid * rows_per_tile + z * ZR, ZR)],
                      msem)
                  for z in range(rows_per_tile // ZR)]
            for m in range(3):
                col_start(b, m, m)
            rv_start(0, 0)
            rv_start(1, 1)
            col_wait(0)
            gather_start(0)
            col_wait(1)
            gather_start(1)
            for d in zd:
                d.wait()
            plsc.subcore_barrier()

            for u in range(D):
                one(b, u, u, first=True)

            def ring(i4, carry):
                for u in range(D):
                    one(b, D * i4 + u, u)
                return carry
            lax.fori_loop(1, n_iter - 1, ring, 0)

            for u in range(D):
                one(b, steps - D + u, u, last=True)
            scatter_wait((steps - 2) % D)
            scatter_wait((steps - 1) % D)
            plsc.subcore_barrier()

            pend = [None] * D
            for z in range(rows_per_tile // K):
                r = z % D
                if pend[r] is not None:
                    pend[r].wait()
                r0 = sid * rows_per_tile + z * K
                pltpu.sync_copy(acc.at[pl.ds(r0, K)], rows[r])
                pend[r] = pltpu.async_copy(rows[r],
                                           out_h.at[cid, b, pl.ds(r0, K)],
                                           msem)
            for d in pend:
                if d is not None:
                    d.wait()

    return spmm(xf, colf, rowp, valp)


def _mlp_kernel(yp_ref, w1t_ref, b1_ref, w2t_ref, b2_ref, out_ref):
    y = yp_ref[0, 0] + yp_ref[1, 0]
    h = jnp.maximum(
        jnp.dot(y, w1t_ref[...], preferred_element_type=jnp.float32)
        + b1_ref[...], 0.0)
    out_ref[0] = (
        jnp.dot(h, w2t_ref[...], preferred_element_type=jnp.float32)
        + b2_ref[...])


def _mlp(ypart, w1t, b1r, w2t, b2r, *, nb, n_pad, c, c_out, blk=512):
    grid = (nb, n_pad // blk)
    return pl.pallas_call(
        _mlp_kernel,
        grid=grid,
        in_specs=[
            pl.BlockSpec((2, 1, blk, c), lambda b, n: (0, b, n, 0)),
            pl.BlockSpec((c, c), lambda b, n: (0, 0)),
            pl.BlockSpec((1, c), lambda b, n: (0, 0)),
            pl.BlockSpec((c, c_out), lambda b, n: (0, 0)),
            pl.BlockSpec((1, c_out), lambda b, n: (0, 0)),
        ],
        out_specs=pl.BlockSpec((1, blk, c_out), lambda b, n: (b, n, 0)),
        out_shape=jax.ShapeDtypeStruct((nb, n_pad, c_out), jnp.float32),
    )(ypart, w1t, b1r, w2t, b2r)


def kernel(x, A_idx, A_val, W1, b1, W2, b2):
    nb, n, c = x.shape
    e = A_val.shape[0]
    c_out = W2.shape[0]

    n_pad = ((n + NS * 128 - 1) // (NS * 128)) * (NS * 128)
    chunk = NW * K * D
    e_pad = ((e + chunk - 1) // chunk) * chunk
    steps = e_pad // (NW * K)

    row = A_idx[0].astype(jnp.int32)
    col = A_idx[1].astype(jnp.int32)
    pad = e_pad - e
    ar = jnp.arange(pad, dtype=jnp.int32)
    rowp = jnp.concatenate([row, n + ar % (n_pad - n)]
                           ).reshape(NW, steps, K)
    valp = jnp.pad(A_val, (0, pad)).reshape(NW, steps, K)
    colp = jnp.concatenate([col, ar % n])
    colf = (colp[None, :] + (jnp.arange(nb, dtype=jnp.int32) * n)[:, None]
            ).reshape(nb, NW, steps, K)
    xf = x.reshape(nb * n, c)

    ypart = _sc_spmm(xf, colf, rowp, valp, n_pad=n_pad, steps=steps, nb=nb,
                     c=c)

    out = _mlp(ypart, W1.T, b1.reshape(1, -1), W2.T, b2.reshape(1, -1),
               nb=nb, n_pad=n_pad, c=c, c_out=c_out)
    return out[:, :n, :]


# speedup vs baseline: 8.5044x; 1.0530x over previous
import functools

import jax
import jax.numpy as jnp
from jax import lax
from jax.experimental import pallas as pl
from jax.experimental.pallas import tpu as pltpu
from jax.experimental.pallas import tpu_sc as plsc

NC = 2
NS = 16
NW = NC * NS
K = 64
D = 4
ZR = 32


def _sc_spmm(xf, colf, rowp, valp, *, n_pad, steps, nb, c):
    rows_per_tile = n_pad // NS
    n_iter = steps // D

    mesh = plsc.VectorSubcoreMesh(core_axis_name="c", subcore_axis_name="s")

    @functools.partial(
        pl.kernel,
        out_type=jax.ShapeDtypeStruct((NC, nb, n_pad, c), jnp.float32),
        mesh=mesh,
        compiler_params=pltpu.CompilerParams(needs_layout_passes=False),
        scratch_types=[
            [pltpu.VMEM((K, c), jnp.float32)] * D,
            [pltpu.VMEM((K,), jnp.int32)] * D,
            [pltpu.VMEM((K,), jnp.int32)] * D,
            [pltpu.VMEM((K,), jnp.float32)] * D,
            pltpu.VMEM((ZR, c), jnp.float32),
            pltpu.VMEM_SHARED((n_pad, c), jnp.float32),
            [pltpu.SemaphoreType.DMA] * D,
            [pltpu.SemaphoreType.DMA] * D,
            [pltpu.SemaphoreType.DMA] * D,
            [pltpu.SemaphoreType.DMA] * D,
            pltpu.SemaphoreType.DMA,
        ],
    )
    def spmm(xf_h, colf_h, rowp_h, valp_h, out_h, rows, colv, rowv, valv,
             zbuf, acc, gsem, ssem, csem, rvsem, msem):
        cid = lax.axis_index("c")
        sid = lax.axis_index("s")
        tid = cid * NS + sid

        def zrow(i, carry):
            for j in range(c // 16):
                zbuf[i, pl.ds(j * 16, 16)] = jnp.zeros((16,), jnp.float32)
            return carry
        lax.fori_loop(0, ZR, zrow, 0)

        def col_start(b, g, m):
            pltpu.async_copy(colf_h.at[b, tid, g], colv[m], csem[m])

        def col_wait(m):
            pltpu.make_async_copy(colf_h.at[0, 0, 0], colv[m],
                                  csem[m]).wait()

        def rv_start(g, m):
            pltpu.async_copy(rowp_h.at[tid, g], rowv[m], rvsem[m])
            pltpu.async_copy(valp_h.at[tid, g], valv[m], rvsem[m])

        def rv_wait(m):
            pltpu.make_async_copy(rowp_h.at[0, 0], rowv[m], rvsem[m]).wait()
            pltpu.make_async_copy(valp_h.at[0, 0], valv[m], rvsem[m]).wait()

        def gather_start(m):
            pltpu.async_copy(xf_h.at[colv[m]], rows[m], gsem[m])

        def gather_wait(m):
            pltpu.make_async_copy(xf_h.at[colv[0]], rows[m],
                                  gsem[m]).wait()

        def scatter_start(m):
            pltpu.async_copy(rows[m], acc.at[rowv[m]], ssem[m], add=True)

        def scatter_wait(m):
            pltpu.make_async_copy(rows[m], acc.at[rowv[0]], ssem[m]).wait()

        def scale(m):
            @plsc.parallel_loop(0, K, unroll=4)
            def _(e):
                vs = plsc.load_gather(valv[m],
                                      [jnp.full((16,), e, jnp.int32)])
                for j in range(c // 16):
                    rows[m][e, pl.ds(j * 16, 16)] = (
                        rows[m][e, pl.ds(j * 16, 16)] * vs)

        def one(b, g, u, *, first=False, last=False):
            r = u % D
            r2 = (u + 2) % D
            r3 = (u + 3) % D
            live2 = not (last and u >= D - 2)
            live3 = not (last and u >= D - 3)
            if not (first and u < 2):
                scatter_wait(r2)
            if live2:
                col_wait(r2)
                gather_start(r2)
                rv_start(g + 2, r2)
            if live3:
                col_start(b, g + 3, r3)
            rv_wait(r)
            gather_wait(r)
            scale(r)
            scatter_start(r)

        for b in range(nb):
            zd = [pltpu.async_copy(
                      zbuf, acc.at[pl.ds(sid * rows_per_tile + z * ZR, ZR)],
                      msem)
                  for z in range(rows_per_tile // ZR)]
            for m in range(3):
                col_start(b, m, m)
            rv_start(0, 0)
            rv_start(1, 1)
            col_wait(0)
            gather_start(0)
            col_wait(1)
            gather_start(1)
            for d in zd:
                d.wait()
            plsc.subcore_barrier()

            for u in range(D):
                one(b, u, u, first=True)

            def ring(i4, carry):
                for u in range(D):
                    one(b, D * i4 + u, u)
                return carry
            lax.fori_loop(1, n_iter - 1, ring, 0)

            for u in range(D):
                one(b, steps - D + u, u, last=True)
            scatter_wait((steps - 2) % D)
            scatter_wait((steps - 1) % D)
            plsc.subcore_barrier()

            pend = [None] * D
            for z in range(rows_per_tile // K):
                r = z % D
                if pend[r] is not None:
                    pend[r].wait()
                r0 = sid * rows_per_tile + z * K
                pltpu.sync_copy(acc.at[pl.ds(r0, K)], rows[r])
                pend[r] = pltpu.async_copy(rows[r],
                                           out_h.at[cid, b, pl.ds(r0, K)],
                                           msem)
            for d in pend:
                if d is not None:
                    d.wait()

    return spmm(xf, colf, rowp, valp)


def _mlp_kernel(yp_ref, w1t_ref, b1_ref, w2t_ref, b2_ref, out_ref):
    y = yp_ref[0, 0] + yp_ref[1, 0]
    h = jnp.maximum(
        jnp.dot(y, w1t_ref[...], preferred_element_type=jnp.float32)
        + b1_ref[...], 0.0)
    out_ref[0] = (
        jnp.dot(h, w2t_ref[...], preferred_element_type=jnp.float32)
        + b2_ref[...])


def _mlp(ypart, w1t, b1r, w2t, b2r, *, nb, n_pad, c, c_out, blk=512):
    grid = (nb, n_pad // blk)
    return pl.pallas_call(
        _mlp_kernel,
        grid=grid,
        in_specs=[
            pl.BlockSpec((2, 1, blk, c), lambda b, n: (0, b, n, 0)),
            pl.BlockSpec((c, c), lambda b, n: (0, 0)),
            pl.BlockSpec((1, c), lambda b, n: (0, 0)),
            pl.BlockSpec((c, c_out), lambda b, n: (0, 0)),
            pl.BlockSpec((1, c_out), lambda b, n: (0, 0)),
        ],
        out_specs=pl.BlockSpec((1, blk, c_out), lambda b, n: (b, n, 0)),
        out_shape=jax.ShapeDtypeStruct((nb, n_pad, c_out), jnp.float32),
    )(ypart, w1t, b1r, w2t, b2r)


def kernel(x, A_idx, A_val, W1, b1, W2, b2):
    nb, n, c = x.shape
    e = A_val.shape[0]
    c_out = W2.shape[0]

    n_pad = ((n + NS * 128 - 1) // (NS * 128)) * (NS * 128)
    chunk = NW * K * D
    e_pad = ((e + chunk - 1) // chunk) * chunk
    steps = e_pad // (NW * K)

    row = A_idx[0].astype(jnp.int32)
    col = A_idx[1].astype(jnp.int32)
    pad = e_pad - e
    ar = jnp.arange(pad, dtype=jnp.int32)
    rowp = jnp.concatenate([row, n + ar % (n_pad - n)]
                           ).reshape(NW, steps, K)
    valp = jnp.pad(A_val, (0, pad)).reshape(NW, steps, K)
    colp = jnp.concatenate([col, ar % n])
    colf = (colp[None, :] + (jnp.arange(nb, dtype=jnp.int32) * n)[:, None]
            ).reshape(nb, NW, steps, K)
    xf = x.reshape(nb * n, c)

    ypart = _sc_spmm(xf, colf, rowp, valp, n_pad=n_pad, steps=steps, nb=nb,
                     c=c)

    out = _mlp(ypart, W1.T, b1.reshape(1, -1), W2.T, b2.reshape(1, -1),
               nb=nb, n_pad=n_pad, c=c, c_out=c_out)
    return out[:, :n, :]


# speedup vs baseline: 9.1405x; 1.0748x over previous
import functools

import jax
import jax.numpy as jnp
from jax import lax
from jax.experimental import pallas as pl
from jax.experimental.pallas import tpu as pltpu
from jax.experimental.pallas import tpu_sc as plsc

NC = 2
NS = 16
NW = NC * NS
K = 64
D = 4
ZR = 32


def _sc_spmm(xf, colf, rowp, valp, *, n_pad, steps, nb, c):
    rows_per_tile = n_pad // NS
    n_iter = steps // D

    mesh = plsc.VectorSubcoreMesh(core_axis_name="c", subcore_axis_name="s")

    @functools.partial(
        pl.kernel,
        out_type=jax.ShapeDtypeStruct((NC, nb, n_pad, c), jnp.float32),
        mesh=mesh,
        compiler_params=pltpu.CompilerParams(needs_layout_passes=False),
        scratch_types=[
            [pltpu.VMEM((K, c), jnp.float32)] * D,
            [pltpu.VMEM((K,), jnp.int32)] * D,
            [pltpu.VMEM((K,), jnp.int32)] * D,
            [pltpu.VMEM((K,), jnp.float32)] * D,
            pltpu.VMEM((ZR, c), jnp.float32),
            pltpu.VMEM_SHARED((n_pad, c), jnp.float32),
            [pltpu.SemaphoreType.DMA] * D,
            [pltpu.SemaphoreType.DMA] * D,
            [pltpu.SemaphoreType.DMA] * D,
            [pltpu.SemaphoreType.DMA] * D,
            pltpu.SemaphoreType.DMA,
        ],
    )
    def spmm(xf_h, colf_h, rowp_h, valp_h, out_h, rows, colv, rowv, valv,
             zbuf, acc, gsem, ssem, csem, rvsem, msem):
        cid = lax.axis_index("c")
        sid = lax.axis_index("s")
        tid = cid * NS + sid

        def zrow(i, carry):
            for j in range(c // 16):
                zbuf[i, pl.ds(j * 16, 16)] = jnp.zeros((16,), jnp.float32)
            return carry
        lax.fori_loop(0, ZR, zrow, 0)

        def col_start(b, g, m):
            del b
            pltpu.async_copy(colf_h.at[tid, g], colv[m], csem[m])

        def col_wait(m):
            pltpu.make_async_copy(colf_h.at[0, 0], colv[m],
                                  csem[m]).wait()

        def rv_start(g, m):
            pltpu.async_copy(rowp_h.at[tid, g], rowv[m], rvsem[m])
            pltpu.async_copy(valp_h.at[tid, g], valv[m], rvsem[m])

        def rv_wait(m):
            pltpu.make_async_copy(rowp_h.at[0, 0], rowv[m], rvsem[m]).wait()
            pltpu.make_async_copy(valp_h.at[0, 0], valv[m], rvsem[m]).wait()

        def gather_start(b, m):
            pltpu.async_copy(xf_h.at[b].at[colv[m]], rows[m], gsem[m])

        def gather_wait(m):
            pltpu.make_async_copy(xf_h.at[0].at[colv[0]], rows[m],
                                  gsem[m]).wait()

        def scatter_start(m):
            pltpu.async_copy(rows[m], acc.at[rowv[m]], ssem[m], add=True)

        def scatter_wait(m):
            pltpu.make_async_copy(rows[m], acc.at[rowv[0]], ssem[m]).wait()

        def scale(m):
            @plsc.parallel_loop(0, K, unroll=4)
            def _(e):
                vs = plsc.load_gather(valv[m],
                                      [jnp.full((16,), e, jnp.int32)])
                for j in range(c // 16):
                    rows[m][e, pl.ds(j * 16, 16)] = (
                        rows[m][e, pl.ds(j * 16, 16)] * vs)

        def one(b, g, u, *, first=False, last=False):
            r = u % D
            r2 = (u + 2) % D
            r3 = (u + 3) % D
            live2 = not (last and u >= D - 2)
            live3 = not (last and u >= D - 3)
            if not (first and u < 2):
                scatter_wait(r2)
            if live2:
                col_wait(r2)
                gather_start(b, r2)
                rv_start(g + 2, r2)
            if live3:
                col_start(b, g + 3, r3)
            rv_wait(r)
            gather_wait(r)
            scale(r)
            scatter_start(r)

        for b in range(nb):
            zd = [pltpu.async_copy(
                      zbuf, acc.at[pl.ds(sid * rows_per_tile + z * ZR, ZR)],
                      msem)
                  for z in range(rows_per_tile // ZR)]
            for m in range(3):
                col_start(b, m, m)
            rv_start(0, 0)
            rv_start(1, 1)
            col_wait(0)
            gather_start(b, 0)
            col_wait(1)
            gather_start(b, 1)
            for d in zd:
                d.wait()
            plsc.subcore_barrier()

            for u in range(D):
                one(b, u, u, first=True)

            def ring(i4, carry):
                for u in range(D):
                    one(b, D * i4 + u, u)
                return carry
            lax.fori_loop(1, n_iter - 1, ring, 0)

            for u in range(D):
                one(b, steps - D + u, u, last=True)
            scatter_wait((steps - 2) % D)
            scatter_wait((steps - 1) % D)
            plsc.subcore_barrier()

            pend = [None] * D
            for z in range(rows_per_tile // K):
                r = z % D
                if pend[r] is not None:
                    pend[r].wait()
                r0 = sid * rows_per_tile + z * K
                pltpu.sync_copy(acc.at[pl.ds(r0, K)], rows[r])
                pend[r] = pltpu.async_copy(rows[r],
                                           out_h.at[cid, b, pl.ds(r0, K)],
                                           msem)
            for d in pend:
                if d is not None:
                    d.wait()

    return spmm(xf, colf, rowp, valp)


def _mlp_kernel(yp_ref, w1t_ref, b1_ref, w2t_ref, b2_ref, out_ref):
    y = yp_ref[0, 0] + yp_ref[1, 0]
    h = jnp.maximum(
        jnp.dot(y, w1t_ref[...], preferred_element_type=jnp.float32)
        + b1_ref[...], 0.0)
    out_ref[0] = (
        jnp.dot(h, w2t_ref[...], preferred_element_type=jnp.float32)
        + b2_ref[...])


def _mlp(ypart, w1t, b1r, w2t, b2r, *, nb, n, c, c_out, blk=1000):
    grid = (nb, n // blk)
    return pl.pallas_call(
        _mlp_kernel,
        grid=grid,
        in_specs=[
            pl.BlockSpec((2, 1, blk, c), lambda b, n: (0, b, n, 0)),
            pl.BlockSpec((c, c), lambda b, n: (0, 0)),
            pl.BlockSpec((1, c), lambda b, n: (0, 0)),
            pl.BlockSpec((c, c_out), lambda b, n: (0, 0)),
            pl.BlockSpec((1, c_out), lambda b, n: (0, 0)),
        ],
        out_specs=pl.BlockSpec((1, blk, c_out), lambda b, n: (b, n, 0)),
        out_shape=jax.ShapeDtypeStruct((nb, n, c_out), jnp.float32),
    )(ypart, w1t, b1r, w2t, b2r)


def kernel(x, A_idx, A_val, W1, b1, W2, b2):
    nb, n, c = x.shape
    e = A_val.shape[0]
    c_out = W2.shape[0]

    n_pad = ((n + NS * 128 - 1) // (NS * 128)) * (NS * 128)
    chunk = NW * K * D
    e_pad = ((e + chunk - 1) // chunk) * chunk
    steps = e_pad // (NW * K)

    row = A_idx[0].astype(jnp.int32)
    col = A_idx[1].astype(jnp.int32)
    pad = e_pad - e
    ar = jnp.arange(pad, dtype=jnp.int32)
    rowp = jnp.concatenate([row, n + ar % (n_pad - n)]
                           ).reshape(NW, steps, K)
    valp = jnp.pad(A_val, (0, pad)).reshape(NW, steps, K)
    colf = jnp.concatenate([col, ar % n]).reshape(NW, steps, K)

    ypart = _sc_spmm(x, colf, rowp, valp, n_pad=n_pad, steps=steps, nb=nb,
                     c=c)

    return _mlp(ypart, W1.T, b1.reshape(1, -1), W2.T, b2.reshape(1, -1),
                nb=nb, n=n, c=c, c_out=c_out)
